# baseline (speedup 1.0000x reference)
import functools

import jax
import jax.numpy as jnp
from jax.experimental import pallas as pl
from jax.experimental.pallas import tpu as pltpu


_COMPILER_PARAMS = pltpu.CompilerParams(
    dimension_semantics=("parallel",),
    vmem_limit_bytes=64 * 1024 * 1024,
)


def _fused_kernel(x_ref, w1_ref, b1_ref, w2_ref, b2_ref,
                  f1_ref, f1b_ref, f2_ref, f2b_ref, f3_ref, f3b_ref,
                  h2_ref, out_ref, *, bsz):
    B = bsz
    blocks = []
    for q in range(4):
        y4 = jnp.dot(x_ref[:, q * 64:(q + 1) * 64, :].reshape(B * 64, 64),
                     w1_ref[...], preferred_element_type=jnp.float32)
        y4 = y4.reshape(B, 64, 128)
        y = jnp.maximum(jnp.maximum(y4[:, :, 0:32], y4[:, :, 32:64]),
                        jnp.maximum(y4[:, :, 64:96], y4[:, :, 96:128]))
        blocks.append(
            jnp.maximum(y + b1_ref[...], 0.0).astype(jnp.bfloat16))
    x2 = jnp.concatenate(blocks, axis=2)

    xc2 = jnp.concatenate(
        [x2[:, s:s + 48, :] for s in (0, 1, 8, 9)], axis=2)
    z4 = jnp.dot(xc2.reshape(B * 48, 512), w2_ref[...],
                 preferred_element_type=jnp.float32)
    z4 = z4.reshape(B, 48, 256)
    z = jnp.maximum(jnp.maximum(z4[:, :, 0:64], z4[:, :, 64:128]),
                    jnp.maximum(z4[:, :, 128:192], z4[:, :, 192:256]))
    y2 = jnp.maximum(z + b2_ref[...], 0.0).astype(jnp.bfloat16)

    flat = y2.reshape(B, 6, 8, 64)[:, :, :6, :].reshape(B, 2304)
    h1 = jnp.dot(flat, f1_ref[...], preferred_element_type=jnp.float32)
    h1 = jnp.maximum(h1 + f1b_ref[...], 0.0)
    h2 = jnp.dot(h1, f2_ref[...], preferred_element_type=jnp.float32)
    h2 = jnp.maximum(h2 + f2b_ref[...], 0.0)
    out = jnp.dot(h2, f3_ref[...], preferred_element_type=jnp.float32) + f3b_ref[...]
    h2_ref[...] = h2
    out_ref[...] = out


def _pack_conv1_input(x_nchw):
    n = x_nchw.shape[0]
    x = jnp.transpose(x_nchw.astype(jnp.bfloat16), (0, 2, 3, 1))
    xp = x.reshape(n, 16, 2, 16, 2, 3).transpose(0, 1, 3, 2, 4, 5)
    xp = xp.reshape(n, 256, 12)
    xp1 = jnp.concatenate([xp, jnp.zeros((n, 1, 12), jnp.bfloat16)], axis=1)
    x_cat = jnp.concatenate(
        [jax.lax.slice_in_dim(xp1, s, s + 240, axis=1) for s in (0, 1, 16, 17)],
        axis=2)
    x_cat = jnp.pad(x_cat, ((0, 0), (0, 16), (0, 16)))
    x_cat = x_cat.reshape(n, 8, 2, 8, 2, 64)
    x_cat = x_cat.transpose(0, 2, 4, 1, 3, 5)
    return x_cat.reshape(n, 256, 64)


def kernel(conv1_wg, conv1_b, conv2_wg, conv2_b, fc1_wm, fc1_b,
           fc2_wm, fc2_b, fc3_wm, fc3_b, x_nchw):
    n = x_nchw.shape[0]
    bsz = 64 if n % 64 == 0 else (16 if n % 16 == 0 else 1)
    x_cat = _pack_conv1_input(x_nchw)
    w1 = jnp.pad(conv1_wg.reshape(48, 128), ((0, 16), (0, 0)))
    w2 = conv2_wg.reshape(512, 256)
    h2, out = pl.pallas_call(
        functools.partial(_fused_kernel, bsz=bsz),
        out_shape=(jax.ShapeDtypeStruct((n, 84), jnp.float32),
                   jax.ShapeDtypeStruct((n, 10), jnp.float32)),
        grid=(n // bsz,),
        in_specs=[
            pl.BlockSpec((bsz, 256, 64), lambda i: (i, 0, 0)),
            pl.BlockSpec((64, 128), lambda i: (0, 0)),
            pl.BlockSpec((1, 32), lambda i: (0, 0)),
            pl.BlockSpec((512, 256), lambda i: (0, 0)),
            pl.BlockSpec((1, 64), lambda i: (0, 0)),
            pl.BlockSpec((2304, 128), lambda i: (0, 0)),
            pl.BlockSpec((1, 128), lambda i: (0, 0)),
            pl.BlockSpec((128, 84), lambda i: (0, 0)),
            pl.BlockSpec((1, 84), lambda i: (0, 0)),
            pl.BlockSpec((84, 10), lambda i: (0, 0)),
            pl.BlockSpec((1, 10), lambda i: (0, 0)),
        ],
        out_specs=(pl.BlockSpec((bsz, 84), lambda i: (i, 0)),
                   pl.BlockSpec((bsz, 10), lambda i: (i, 0))),
        compiler_params=_COMPILER_PARAMS,
    )(x_cat, w1, conv1_b, w2, conv2_b,
      fc1_wm, fc1_b, fc2_wm, fc2_b, fc3_wm, fc3_b)
    return h2, out


# speedup vs baseline: 1.0060x; 1.0060x over previous
import functools

import jax
import jax.numpy as jnp
from jax.experimental import pallas as pl
from jax.experimental.pallas import tpu as pltpu


_COMPILER_PARAMS = pltpu.CompilerParams(
    dimension_semantics=("parallel",),
    vmem_limit_bytes=64 * 1024 * 1024,
)


def _fused_kernel(x_ref, w1_ref, b1_ref, w2_ref, b2_ref,
                  f1_ref, f1b_ref, f2_ref, f2b_ref, f3_ref, f3b_ref,
                  h2_ref, out_ref, *, bsz):
    B = bsz
    blocks = []
    for q in range(4):
        y4 = jnp.dot(x_ref[:, q * 64:(q + 1) * 64, :].reshape(B * 64, 64),
                     w1_ref[...], preferred_element_type=jnp.float32)
        y4 = y4.reshape(B, 64, 128)
        y = jnp.maximum(jnp.maximum(y4[:, :, 0:32], y4[:, :, 32:64]),
                        jnp.maximum(y4[:, :, 64:96], y4[:, :, 96:128]))
        blocks.append(
            jnp.maximum(y + b1_ref[...], 0.0).astype(jnp.bfloat16))
    x2 = jnp.concatenate(blocks, axis=2)

    xc2 = jnp.concatenate(
        [x2[:, s:s + 48, :] for s in (0, 1, 8, 9)], axis=2)
    z4 = jnp.dot(xc2.reshape(B * 48, 512), w2_ref[...],
                 preferred_element_type=jnp.float32)
    z4 = z4.reshape(B, 48, 256)
    z = jnp.maximum(jnp.maximum(z4[:, :, 0:64], z4[:, :, 64:128]),
                    jnp.maximum(z4[:, :, 128:192], z4[:, :, 192:256]))
    y2 = jnp.maximum(z + b2_ref[...], 0.0).astype(jnp.bfloat16)

    flat = y2.reshape(B, 6, 8, 64)[:, :, :6, :].reshape(B, 2304)
    h1 = jnp.dot(flat, f1_ref[...], preferred_element_type=jnp.float32)
    h1 = jnp.maximum(h1 + f1b_ref[...], 0.0)
    h2 = jnp.dot(h1, f2_ref[...], preferred_element_type=jnp.float32)
    h2 = jnp.maximum(h2 + f2b_ref[...], 0.0)
    out = jnp.dot(h2, f3_ref[...], preferred_element_type=jnp.float32) + f3b_ref[...]
    h2_ref[...] = h2
    out_ref[...] = out


def _pack_conv1_input(x_nchw):
    n = x_nchw.shape[0]
    x = jnp.transpose(x_nchw, (0, 2, 3, 1))
    xp = x.reshape(n, 16, 2, 16, 2, 3).transpose(0, 1, 3, 2, 4, 5)
    xp = xp.reshape(n, 256, 12)
    xp1 = jnp.concatenate([xp, jnp.zeros((n, 1, 12), jnp.float32)], axis=1)
    x_cat = jnp.concatenate(
        [jax.lax.slice_in_dim(xp1, s, s + 240, axis=1) for s in (0, 1, 16, 17)],
        axis=2)
    x_cat = jnp.pad(x_cat, ((0, 0), (0, 16), (0, 16)))
    x_cat = x_cat.reshape(n, 8, 2, 8, 2, 64)
    x_cat = x_cat.transpose(0, 2, 4, 1, 3, 5)
    return x_cat.reshape(n, 256, 64).astype(jnp.bfloat16)


def kernel(conv1_wg, conv1_b, conv2_wg, conv2_b, fc1_wm, fc1_b,
           fc2_wm, fc2_b, fc3_wm, fc3_b, x_nchw):
    n = x_nchw.shape[0]
    bsz = 128 if n % 128 == 0 else (16 if n % 16 == 0 else 1)
    x_cat = _pack_conv1_input(x_nchw)
    w1 = jnp.pad(conv1_wg.reshape(48, 128), ((0, 16), (0, 0)))
    w2 = conv2_wg.reshape(512, 256)
    h2, out = pl.pallas_call(
        functools.partial(_fused_kernel, bsz=bsz),
        out_shape=(jax.ShapeDtypeStruct((n, 84), jnp.float32),
                   jax.ShapeDtypeStruct((n, 10), jnp.float32)),
        grid=(n // bsz,),
        in_specs=[
            pl.BlockSpec((bsz, 256, 64), lambda i: (i, 0, 0)),
            pl.BlockSpec((64, 128), lambda i: (0, 0)),
            pl.BlockSpec((1, 32), lambda i: (0, 0)),
            pl.BlockSpec((512, 256), lambda i: (0, 0)),
            pl.BlockSpec((1, 64), lambda i: (0, 0)),
            pl.BlockSpec((2304, 128), lambda i: (0, 0)),
            pl.BlockSpec((1, 128), lambda i: (0, 0)),
            pl.BlockSpec((128, 84), lambda i: (0, 0)),
            pl.BlockSpec((1, 84), lambda i: (0, 0)),
            pl.BlockSpec((84, 10), lambda i: (0, 0)),
            pl.BlockSpec((1, 10), lambda i: (0, 0)),
        ],
        out_specs=(pl.BlockSpec((bsz, 84), lambda i: (i, 0)),
                   pl.BlockSpec((bsz, 10), lambda i: (i, 0))),
        compiler_params=_COMPILER_PARAMS,
    )(x_cat, w1, conv1_b, w2, conv2_b,
      fc1_wm, fc1_b, fc2_wm, fc2_b, fc3_wm, fc3_b)
    return h2, out
